# baseline (speedup 1.0000x reference)
import functools

import jax
import jax.numpy as jnp
from jax import lax
from jax.experimental import pallas as pl
from jax.experimental.pallas import tpu as pltpu
from jax.experimental.pallas import tpu_sc as plsc

_NUM_FIELDS = 26
_VOCAB = 100000
_HIDDEN = 32
_BATCH = 16384

_SPLIT = 2
_F = _NUM_FIELDS // _SPLIT
_R = _BATCH * _F
_CH = 832
_LANES = 16


def _make_sc_gather():
    info = plsc.get_sparse_core_info()
    nw = info.num_cores * info.num_subcores
    assert _R % (nw * _CH) == 0
    nch = _R // (nw * _CH)
    vecs = _CH // _LANES

    mesh = plsc.VectorSubcoreMesh(core_axis_name="c", subcore_axis_name="s")

    @functools.partial(
        pl.kernel,
        mesh=mesh,
        out_type=jax.ShapeDtypeStruct((_R, _HIDDEN), jnp.float32),
        scratch_types=[
            pltpu.VMEM((nch, _CH), jnp.int32),
            pltpu.VMEM((_CH, _HIDDEN), jnp.float32),
            pltpu.VMEM((_CH, _HIDDEN), jnp.float32),
            pltpu.SemaphoreType.DMA,
            pltpu.SemaphoreType.DMA,
            pltpu.SemaphoreType.DMA,
            pltpu.SemaphoreType.DMA,
        ],
        compiler_params=pltpu.CompilerParams(use_tc_tiling_on_sc=False),
    )
    def sc_gather(x_hbm, tab_hbm, out_hbm, idx_v, rows_a, rows_b, ga, gb,
                  wa, wb):
        wid = lax.axis_index("s") * info.num_cores + lax.axis_index("c")
        chunk0 = wid * nch

        pltpu.sync_copy(x_hbm.at[pl.ds(chunk0, nch)], idx_v)

        def fix_chunk(c, _):
            def fix_vec(v, _):
                g0 = (chunk0 + c) * _CH + v * _LANES
                fld = (g0 + lax.iota(jnp.int32, _LANES)) % _F
                sl = pl.ds(v * _LANES, _LANES)
                idx_v[c, sl] = idx_v[c, sl] + fld * _VOCAB
                return 0
            return lax.fori_loop(0, vecs, fix_vec, 0)
        lax.fori_loop(0, nch, fix_chunk, 0)

        def g_start(c, buf, sem):
            pltpu.make_async_copy(tab_hbm.at[idx_v.at[c]], buf, sem).start()

        def g_wait(buf, sem):
            pltpu.make_async_copy(tab_hbm.at[idx_v.at[0]], buf, sem).wait()

        def w_start(c, buf, sem):
            dst = out_hbm.at[pl.ds((chunk0 + c) * _CH, _CH)]
            pltpu.make_async_copy(buf, dst, sem).start()

        def w_wait(c, buf, sem):
            dst = out_hbm.at[pl.ds((chunk0 + c) * _CH, _CH)]
            pltpu.make_async_copy(buf, dst, sem).wait()

        g_start(0, rows_a, ga)
        npairs = nch // 2

        def pair(i, _):
            c0 = 2 * i
            g_start(c0 + 1, rows_b, gb)
            g_wait(rows_a, ga)
            w_start(c0, rows_a, wa)
            g_wait(rows_b, gb)
            w_start(c0 + 1, rows_b, wb)
            w_wait(c0, rows_a, wa)

            @pl.when(c0 + 2 < nch)
            def _():
                g_start(c0 + 2, rows_a, ga)
            w_wait(c0 + 1, rows_b, wb)
            return 0

        lax.fori_loop(0, npairs, pair, 0)

        if nch % 2:
            c = nch - 1
            g_wait(rows_a, ga)
            w_start(c, rows_a, wa)
            w_wait(c, rows_a, wa)

    return sc_gather


_sc_gather = _make_sc_gather()


def kernel(x_n_cat, tables):
    x32 = x_n_cat.astype(jnp.int32)
    pieces = []
    for s in range(_SPLIT):
        x2d = x32[:, s * _F:(s + 1) * _F].reshape(_R // _CH, _CH)
        tab2d = tables[s * _F:(s + 1) * _F].reshape(_F * _VOCAB, _HIDDEN)
        out = _sc_gather(x2d, tab2d)
        pieces.append(out.reshape(_BATCH, _F * _HIDDEN))
    return jnp.concatenate(pieces, axis=1)


# speedup vs baseline: 1.5351x; 1.5351x over previous
import functools

import jax
import jax.numpy as jnp
from jax import lax
from jax.experimental import pallas as pl
from jax.experimental.pallas import tpu as pltpu
from jax.experimental.pallas import tpu_sc as plsc

_NUM_FIELDS = 26
_VOCAB = 100000
_HIDDEN = 32
_BATCH = 16384

_SPLIT = 1
_F = _NUM_FIELDS // _SPLIT
_R = _BATCH * _F
_CH = 1024
_LANES = 16


def _make_sc_gather():
    info = plsc.get_sparse_core_info()
    nw = info.num_cores * info.num_subcores
    assert _R % (nw * _CH) == 0
    nch = _R // (nw * _CH)
    vecs = _CH // _LANES

    mesh = plsc.VectorSubcoreMesh(core_axis_name="c", subcore_axis_name="s")

    @functools.partial(
        pl.kernel,
        mesh=mesh,
        out_type=jax.ShapeDtypeStruct((_R, _HIDDEN), jnp.float32),
        scratch_types=[
            pltpu.VMEM((nch, _CH), jnp.int32),
            pltpu.VMEM((_CH, _HIDDEN), jnp.float32),
            pltpu.VMEM((_CH, _HIDDEN), jnp.float32),
            pltpu.SemaphoreType.DMA,
            pltpu.SemaphoreType.DMA,
            pltpu.SemaphoreType.DMA,
            pltpu.SemaphoreType.DMA,
        ],
        compiler_params=pltpu.CompilerParams(use_tc_tiling_on_sc=False),
    )
    def sc_gather(x_hbm, tab_hbm, out_hbm, idx_v, rows_a, rows_b, ga, gb,
                  wa, wb):
        wid = lax.axis_index("s") * info.num_cores + lax.axis_index("c")
        chunk0 = wid * nch

        pltpu.sync_copy(x_hbm.at[pl.ds(chunk0, nch)], idx_v)

        def fix_chunk(c, _):
            def fix_vec(v, _):
                g0 = (chunk0 + c) * _CH + v * _LANES
                fld = (g0 + lax.iota(jnp.int32, _LANES)) % _F
                sl = pl.ds(v * _LANES, _LANES)
                idx_v[c, sl] = idx_v[c, sl] + fld * _VOCAB
                return 0
            return lax.fori_loop(0, vecs, fix_vec, 0)
        lax.fori_loop(0, nch, fix_chunk, 0)

        def g_start(c, buf, sem):
            pltpu.make_async_copy(tab_hbm.at[idx_v.at[c]], buf, sem).start()

        def g_wait(buf, sem):
            pltpu.make_async_copy(tab_hbm.at[idx_v.at[0]], buf, sem).wait()

        def w_start(c, buf, sem):
            dst = out_hbm.at[pl.ds((chunk0 + c) * _CH, _CH)]
            pltpu.make_async_copy(buf, dst, sem).start()

        def w_wait(c, buf, sem):
            dst = out_hbm.at[pl.ds((chunk0 + c) * _CH, _CH)]
            pltpu.make_async_copy(buf, dst, sem).wait()

        g_start(0, rows_a, ga)
        npairs = nch // 2

        def pair(i, _):
            c0 = 2 * i
            g_start(c0 + 1, rows_b, gb)
            g_wait(rows_a, ga)
            w_start(c0, rows_a, wa)
            g_wait(rows_b, gb)
            w_start(c0 + 1, rows_b, wb)
            w_wait(c0, rows_a, wa)

            @pl.when(c0 + 2 < nch)
            def _():
                g_start(c0 + 2, rows_a, ga)
            w_wait(c0 + 1, rows_b, wb)
            return 0

        lax.fori_loop(0, npairs, pair, 0)

        if nch % 2:
            c = nch - 1
            g_wait(rows_a, ga)
            w_start(c, rows_a, wa)
            w_wait(c, rows_a, wa)

    return sc_gather


_sc_gather = _make_sc_gather()


def kernel(x_n_cat, tables):
    x32 = x_n_cat.astype(jnp.int32)
    pieces = []
    for s in range(_SPLIT):
        x2d = x32[:, s * _F:(s + 1) * _F].reshape(_R // _CH, _CH)
        tab2d = tables[s * _F:(s + 1) * _F].reshape(_F * _VOCAB, _HIDDEN)
        out = _sc_gather(x2d, tab2d)
        pieces.append(out.reshape(_BATCH, _F * _HIDDEN))
    return jnp.concatenate(pieces, axis=1)
